# baseline (speedup 1.0000x reference)
import math
import numpy as np

import jax
import jax.numpy as jnp
from jax.experimental import pallas as pl
from jax.experimental.pallas import tpu as pltpu

LMAX = 2
MMAX = 2
K = (LMAX + 1) ** 2
SPHERE_CH = 8
HIDDEN_CH = 8
NUM_HEADS = 2
ATTN_ALPHA_CH = 4
ATTN_VALUE_CH = 4
OUTPUT_CH = 8
SILU_SCALE = 1.0 / 0.6

C_IN1 = 2 * SPHERE_CH
ALPHA_TOT = NUM_HEADS * ATTN_ALPHA_CH
VALUE_TOT = NUM_HEADS * ATTN_VALUE_CH
EXTRA_M0 = ALPHA_TOT + HIDDEN_CH

EDGE_TILE = 512

MSG_W = K * VALUE_TOT
PACK_W = 128
PAD_W = PACK_W - MSG_W - NUM_HEADS

PROJ_W = K * OUTPUT_CH
PROJ_PACK_W = 128

NODE_W = 128
X_W = K * C_IN1

M_IDX = [([l * l + l for l in range(LMAX + 1)], [])]
for _m in range(1, MMAX + 1):
    M_IDX.append(([l * l + l + _m for l in range(_m, LMAX + 1)],
                  [l * l + l - _m for l in range(_m, LMAX + 1)]))

L_PER_COEF = np.concatenate([[l] * (2 * l + 1) for l in range(LMAX + 1)]).astype(np.int32)


def _expansion_mats():
    prot = np.zeros((K, K * K, X_W), np.float32)
    trot = np.zeros((K, X_W, X_W), np.float32)
    pinv = np.zeros((K, K * K, MSG_W), np.float32)
    tinv = np.zeros((K, MSG_W, MSG_W), np.float32)
    for j in range(K):
        for k in range(K):
            prot[j, j * K + k, k * C_IN1:(k + 1) * C_IN1] = 1.0
            pinv[j, j * K + k, k * VALUE_TOT:(k + 1) * VALUE_TOT] = 1.0
            for c in range(SPHERE_CH):
                trot[j, j * SPHERE_CH + c, k * C_IN1 + c] = 1.0
                trot[j, K * SPHERE_CH + j * SPHERE_CH + c,
                     k * C_IN1 + SPHERE_CH + c] = 1.0
            for c in range(VALUE_TOT):
                tinv[j, j * VALUE_TOT + c, k * VALUE_TOT + c] = 1.0
    return prot, trot, pinv, tinv


_PROT, _TROT, _PINV, _TINV = _expansion_mats()


def _scaled_silu(x):
    return x * jax.nn.sigmoid(x) * SILU_SCALE


def _layer_norm(x, g, b, eps=1e-5):
    mu = jnp.mean(x, axis=-1, keepdims=True)
    var = jnp.mean((x - mu) ** 2, axis=-1, keepdims=True)
    return (x - mu) * jax.lax.rsqrt(var + eps) * g + b


def _smooth_leaky_relu(x, alpha=0.2):
    return ((1.0 + alpha) / 2.0) * x + ((1.0 - alpha) / 2.0) * x * (2.0 * jax.nn.sigmoid(x) - 1.0)


def _so2_conv_coefs(coefs, w_list, b0, c_in, m_out, rad=None, extra=0):
    out = [None] * K
    f32 = jnp.float32
    idx0 = M_IDX[0][0]
    x0 = jnp.concatenate([coefs[i] for i in idx0], axis=-1)
    off = len(idx0) * c_in
    if rad is not None:
        x0 = x0 * rad[:, :off]
    y0 = jnp.dot(x0, w_list[0], preferred_element_type=f32) + b0
    x_extra = None
    if extra:
        x_extra = y0[:, :extra]
        y0 = y0[:, extra:]
    for t, i in enumerate(idx0):
        out[i] = y0[:, t * m_out:(t + 1) * m_out]
    for m in range(1, MMAX + 1):
        plus_idx, minus_idx = M_IDX[m]
        nm = len(plus_idx)
        in_w = nm * c_in
        half = nm * m_out
        xp = jnp.concatenate([coefs[i] for i in plus_idx], axis=-1)
        xm = jnp.concatenate([coefs[i] for i in minus_idx], axis=-1)
        if rad is not None:
            r = rad[:, off:off + in_w]
            xp = xp * r
            xm = xm * r
        off += in_w
        yp = jnp.dot(xp, w_list[m], preferred_element_type=f32)
        ym = jnp.dot(xm, w_list[m], preferred_element_type=f32)
        op = yp[:, :half] - ym[:, half:]
        om = yp[:, half:] + ym[:, :half]
        for t, i in enumerate(plus_idx):
            out[i] = op[:, t * m_out:(t + 1) * m_out]
        for t, i in enumerate(minus_idx):
            out[i] = om[:, t * m_out:(t + 1) * m_out]
    return out, x_extra


def _s2_act_coefs(coefs, gating, tg_exp, fg_exp, ch):
    x = jnp.concatenate(coefs, axis=-1)
    g = _scaled_silu(jnp.dot(x, tg_exp, preferred_element_type=jnp.float32))
    y = jnp.dot(g, fg_exp, preferred_element_type=jnp.float32)
    out = [_scaled_silu(gating)]
    for k in range(1, K):
        out.append(y[:, k * ch:(k + 1) * ch])
    return out


def _fused_edge_kernel(
        idx_ref, ed_ref, wig_ref, wiginv_ref, ntab_ref,
        rw1_ref, rb1_ref, rg1_ref, rbe1_ref,
        rw2_ref, rb2_ref, rg2_ref, rbe2_ref,
        rw3_ref, rb3_ref,
        c1w0_ref, c1b0_ref, c1w1_ref, c1w2_ref,
        tgx_ref, fgx_ref,
        c2w0_ref, c2b0_ref, c2w1_ref, c2w2_ref,
        ag_ref, ab_ref, adot_ref,
        prot_ref, trot_ref, pinv_ref, tinv_ref,
        out_ref, src_rows, tgt_rows):
    f32 = jnp.float32
    te = EDGE_TILE

    for mi in range(te):
        src_rows[mi] = ntab_ref[idx_ref[0, 0, mi], 0]
        tgt_rows[mi] = ntab_ref[idx_ref[0, 1, mi], 0]
    xs = src_rows[...]
    xt = tgt_rows[...]

    x_edge = jnp.concatenate(
        [ed_ref[...], xs[:, 72:88], xt[:, 88:104]], axis=1)
    h = jnp.dot(x_edge, rw1_ref[...], preferred_element_type=f32) + rb1_ref[...]
    h = _scaled_silu(_layer_norm(h, rg1_ref[...], rbe1_ref[...]))
    h = jnp.dot(h, rw2_ref[...], preferred_element_type=f32) + rb2_ref[...]
    h = _scaled_silu(_layer_norm(h, rg2_ref[...], rbe2_ref[...]))
    rad = jnp.dot(h, rw3_ref[...], preferred_element_type=f32) + rb3_ref[...]

    X = jnp.concatenate([xs[:, :72], xt[:, :72]], axis=1)
    wigi = wiginv_ref[...]
    rot_cat = None
    for j in range(K):
        d = jnp.dot(wigi, prot_ref[j], preferred_element_type=f32)
        v = jnp.dot(X, trot_ref[j], preferred_element_type=f32)
        rot_cat = d * v if rot_cat is None else rot_cat + d * v
    rot = [rot_cat[:, i * C_IN1:(i + 1) * C_IN1] for i in range(K)]

    hid, extra = _so2_conv_coefs(
        rot, [c1w0_ref[...], c1w1_ref[...], c1w2_ref[...]], c1b0_ref[...],
        c_in=C_IN1, m_out=HIDDEN_CH, rad=rad, extra=EXTRA_M0)
    alpha_feat = extra[:, :ALPHA_TOT]
    gating = extra[:, ALPHA_TOT:]

    act = _s2_act_coefs(hid, gating, tgx_ref[...], fgx_ref[...], HIDDEN_CH)

    val, _ = _so2_conv_coefs(
        act, [c2w0_ref[...], c2w1_ref[...], c2w2_ref[...]], c2b0_ref[...],
        c_in=HIDDEN_CH, m_out=VALUE_TOT, rad=None, extra=0)

    adot = adot_ref[...]
    cols = []
    for hd in range(NUM_HEADS):
        a = alpha_feat[:, hd * ATTN_ALPHA_CH:(hd + 1) * ATTN_ALPHA_CH]
        a = _smooth_leaky_relu(_layer_norm(a, ag_ref[...], ab_ref[...]))
        cols.append(jnp.sum(a * adot[hd:hd + 1, :], axis=-1, keepdims=True))
    alpha = jnp.concatenate(cols, axis=-1)

    vcat = jnp.concatenate(val, axis=1)
    wigf = wig_ref[...]
    msg = None
    for j in range(K):
        d = jnp.dot(wigf, pinv_ref[j], preferred_element_type=f32)
        v = jnp.dot(vcat, tinv_ref[j], preferred_element_type=f32)
        msg = d * v if msg is None else msg + d * v

    w = jnp.exp(alpha)
    parts = []
    for hd in range(NUM_HEADS):
        col = w[:, hd:hd + 1]
        parts.append(jnp.broadcast_to(col, (te, ATTN_VALUE_CH)))
    block = jnp.concatenate(parts, axis=1)
    wfull = jnp.concatenate([block] * K, axis=1)

    out_ref[...] = jnp.concatenate(
        [msg * wfull, w, jnp.zeros((te, PAD_W), f32)], axis=-1)


def _fused_edge_messages(idx2, edge_distance, wig2, wiginv2, node_tab, weights):
    E = wig2.shape[0]
    te = EDGE_TILE
    ns = E // te // 2
    grid = (2, ns)

    def row_spec(width):
        return pl.BlockSpec((te, width), lambda c, i: (c * ns + i, 0))

    in_specs = [
        pl.BlockSpec((1, 2, te), lambda c, i: (c * ns + i, 0, 0),
                     memory_space=pltpu.SMEM),
        row_spec(edge_distance.shape[1]),
        row_spec(wig2.shape[1]), row_spec(wiginv2.shape[1]),
        pl.BlockSpec(node_tab.shape, lambda c, i: (0, 0, 0)),
    ]
    in_specs += [pl.BlockSpec(w.shape, lambda c, i, n=w.ndim: (0,) * n)
                 for w in weights]

    out_shape = jax.ShapeDtypeStruct((E, PACK_W), jnp.float32)
    out_specs = pl.BlockSpec((te, PACK_W), lambda c, i: (c * ns + i, 0))

    return pl.pallas_call(
        _fused_edge_kernel,
        out_shape=out_shape,
        grid=grid,
        in_specs=in_specs,
        out_specs=out_specs,
        scratch_shapes=[pltpu.VMEM((te, NODE_W), jnp.float32),
                        pltpu.VMEM((te, NODE_W), jnp.float32)],
        compiler_params=pltpu.CompilerParams(
            dimension_semantics=("parallel", "arbitrary"),
            vmem_limit_bytes=96 * 1024 * 1024),
    )(idx2, edge_distance, wig2, wiginv2, node_tab, *weights)


def _node_proj_kernel(acc_ref, w_ref, b_ref, o_ref):
    acc = acc_ref[...]
    x = acc[:, :MSG_W]
    z = acc[:, MSG_W:MSG_W + NUM_HEADS]
    inv = 1.0 / (z + 1e-16)
    parts = []
    for hd in range(NUM_HEADS):
        col = inv[:, hd:hd + 1]
        parts.append(jnp.broadcast_to(col, (col.shape[0], ATTN_VALUE_CH)))
    block = jnp.concatenate(parts, axis=1)
    inv_full = jnp.concatenate([block] * K, axis=1)
    o_ref[...] = jnp.dot(x * inv_full, w_ref[...],
                         preferred_element_type=jnp.float32) + b_ref[...]


def _node_divide_project(acc, wbd_pad, bias_row):
    N = acc.shape[0]
    return pl.pallas_call(
        _node_proj_kernel,
        out_shape=jax.ShapeDtypeStruct((N, PROJ_PACK_W), jnp.float32),
        grid=(1,),
        in_specs=[pl.BlockSpec((N, PACK_W), lambda i: (0, 0)),
                  pl.BlockSpec((MSG_W, PROJ_PACK_W), lambda i: (0, 0)),
                  pl.BlockSpec((1, PROJ_PACK_W), lambda i: (0, 0))],
        out_specs=pl.BlockSpec((N, PROJ_PACK_W), lambda i: (0, 0)),
    )(acc, wbd_pad, bias_row)


def kernel(x_emb, atomic_numbers, edge_distance, edge_index, wigner, wigner_inv,
           to_grid, from_grid, source_embedding, target_embedding,
           rad1_w1, rad1_b1, rad1_ln1_g, rad1_ln1_b, rad1_w2, rad1_b2,
           rad1_ln2_g, rad1_ln2_b, rad1_w3, rad1_b3,
           conv1_w0, conv1_b0, conv1_w1, conv1_w2,
           conv2_w0, conv2_b0, conv2_w1, conv2_w2,
           alpha_ln_g, alpha_ln_b, alpha_dot, proj_w, proj_b):
    E = edge_index.shape[1]
    N = x_emb.shape[0]
    te = EDGE_TILE
    src, tgt = edge_index[0], edge_index[1]

    node_tab = jnp.concatenate(
        [x_emb.reshape(N, K * SPHERE_CH),
         source_embedding[atomic_numbers],
         target_embedding[atomic_numbers],
         jnp.zeros((N, NODE_W - K * SPHERE_CH - 32), jnp.float32)],
        axis=1).reshape(N, 1, NODE_W)

    idx2 = jnp.stack([src, tgt], axis=0).reshape(2, E // te, te).transpose(1, 0, 2)

    wig2 = wigner.reshape(E, K * K)
    wiginv2 = wigner_inv.reshape(E, K * K)

    eye_h = jnp.eye(HIDDEN_CH, dtype=jnp.float32)
    tg_exp = jnp.kron(to_grid.T, eye_h)
    fg_exp = jnp.kron(from_grid, eye_h)

    _r = lambda v: v.reshape(1, -1)
    weights = [
        rad1_w1, _r(rad1_b1), _r(rad1_ln1_g), _r(rad1_ln1_b),
        rad1_w2, _r(rad1_b2), _r(rad1_ln2_g), _r(rad1_ln2_b),
        rad1_w3, _r(rad1_b3),
        conv1_w0, _r(conv1_b0), conv1_w1, conv1_w2,
        tg_exp, fg_exp,
        conv2_w0, _r(conv2_b0), conv2_w1, conv2_w2,
        _r(alpha_ln_g), _r(alpha_ln_b), alpha_dot,
        jnp.asarray(_PROT), jnp.asarray(_TROT),
        jnp.asarray(_PINV), jnp.asarray(_TINV),
    ]

    packed = _fused_edge_messages(idx2, edge_distance, wig2, wiginv2,
                                  node_tab, weights)

    acc = jax.ops.segment_sum(packed, tgt, num_segments=N)

    w_per = jnp.transpose(proj_w[L_PER_COEF], (0, 2, 1))
    eye_k = jnp.eye(K, dtype=jnp.float32)
    wbd = (eye_k[:, None, :, None] * w_per[:, :, None, :]).reshape(MSG_W, PROJ_W)
    wbd_pad = jnp.zeros((MSG_W, PROJ_PACK_W), jnp.float32).at[:, :PROJ_W].set(wbd)
    bias_row = jnp.zeros((1, PROJ_PACK_W), jnp.float32).at[0, :OUTPUT_CH].set(proj_b)
    out = _node_divide_project(acc, wbd_pad, bias_row)[:, :PROJ_W]
    return out.reshape(N, K, OUTPUT_CH)


# speedup vs baseline: 1.6030x; 1.6030x over previous
import math
import numpy as np

import jax
import jax.numpy as jnp
from jax.experimental import pallas as pl
from jax.experimental.pallas import tpu as pltpu

LMAX = 2
MMAX = 2
K = (LMAX + 1) ** 2
SPHERE_CH = 8
HIDDEN_CH = 8
NUM_HEADS = 2
ATTN_ALPHA_CH = 4
ATTN_VALUE_CH = 4
OUTPUT_CH = 8
SILU_SCALE = 1.0 / 0.6

C_IN1 = 2 * SPHERE_CH
ALPHA_TOT = NUM_HEADS * ATTN_ALPHA_CH
VALUE_TOT = NUM_HEADS * ATTN_VALUE_CH
EXTRA_M0 = ALPHA_TOT + HIDDEN_CH

EDGE_TILE = 512

MSG_W = K * VALUE_TOT
PACK_W = 128
PAD_W = PACK_W - MSG_W - NUM_HEADS

PROJ_W = K * OUTPUT_CH
PROJ_PACK_W = 128

NODE_W = 128
X_W = K * C_IN1

IDX0 = [0, 2, 6]
PLUS1, MINUS1 = [3, 7], [1, 5]
PLUS2, MINUS2 = [8], [4]

L_PER_COEF = np.concatenate([[l] * (2 * l + 1) for l in range(LMAX + 1)]).astype(np.int32)


def _expansion_mats():
    prot = np.zeros((K, K * K, X_W), np.float32)
    trot = np.zeros((K, X_W, X_W), np.float32)
    pinv = np.zeros((K, K * K, MSG_W), np.float32)
    tinv = np.zeros((K, MSG_W, MSG_W), np.float32)
    for j in range(K):
        for k in range(K):
            prot[j, j * K + k, k * C_IN1:(k + 1) * C_IN1] = 1.0
            pinv[j, j * K + k, k * VALUE_TOT:(k + 1) * VALUE_TOT] = 1.0
            for c in range(SPHERE_CH):
                trot[j, j * SPHERE_CH + c, k * C_IN1 + c] = 1.0
                trot[j, K * SPHERE_CH + j * SPHERE_CH + c,
                     k * C_IN1 + SPHERE_CH + c] = 1.0
            for c in range(VALUE_TOT):
                tinv[j, j * VALUE_TOT + c, k * VALUE_TOT + c] = 1.0
    return prot, trot, pinv, tinv


def _sel_t(coefs, ch):
    s = np.zeros((K * ch, len(coefs) * ch), np.float32)
    for t, i in enumerate(coefs):
        for c in range(ch):
            s[i * ch + c, t * ch + c] = 1.0
    return s


def _gather_mat(ch, extra):
    n0, n1, n2 = len(IDX0) * ch, len(PLUS1) * ch, len(PLUS2) * ch
    cols = extra + n0 + 2 * (2 * n1) + 2 * (2 * n2)
    g = np.zeros((cols, K * ch), np.float32)
    o = extra
    for t, i in enumerate(IDX0):
        for c in range(ch):
            g[o + t * ch + c, i * ch + c] = 1.0
    o += n0
    for t, i in enumerate(PLUS1):
        for c in range(ch):
            g[o + t * ch + c, i * ch + c] = 1.0
    o += n1
    for t, i in enumerate(MINUS1):
        for c in range(ch):
            g[o + t * ch + c, i * ch + c] = 1.0
    o += n1
    for t, i in enumerate(MINUS1):
        for c in range(ch):
            g[o + t * ch + c, i * ch + c] = 1.0
    o += n1
    for t, i in enumerate(PLUS1):
        for c in range(ch):
            g[o + t * ch + c, i * ch + c] = -1.0
    o += n1
    for t, i in enumerate(PLUS2):
        for c in range(ch):
            g[o + t * ch + c, i * ch + c] = 1.0
    o += n2
    for t, i in enumerate(MINUS2):
        for c in range(ch):
            g[o + t * ch + c, i * ch + c] = 1.0
    o += n2
    for t, i in enumerate(MINUS2):
        for c in range(ch):
            g[o + t * ch + c, i * ch + c] = 1.0
    o += n2
    for t, i in enumerate(PLUS2):
        for c in range(ch):
            g[o + t * ch + c, i * ch + c] = -1.0
    return g


def _rad_expand():
    r = np.zeros((96, X_W), np.float32)
    for t, i in enumerate(IDX0):
        for c in range(C_IN1):
            r[t * C_IN1 + c, i * C_IN1 + c] = 1.0
    for t in range(len(PLUS1)):
        for c in range(C_IN1):
            r[48 + t * C_IN1 + c, PLUS1[t] * C_IN1 + c] = 1.0
            r[48 + t * C_IN1 + c, MINUS1[t] * C_IN1 + c] = 1.0
    for c in range(C_IN1):
        r[80 + c, PLUS2[0] * C_IN1 + c] = 1.0
        r[80 + c, MINUS2[0] * C_IN1 + c] = 1.0
    return r


_PROT, _TROT, _PINV, _TINV = _expansion_mats()
_G1 = _gather_mat(HIDDEN_CH, EXTRA_M0)
_G2 = _gather_mat(VALUE_TOT, 0)
_RADX = _rad_expand()
_M16 = np.full((16, 16), 1.0 / 16.0, np.float32)
_M8 = np.kron(np.eye(2, dtype=np.float32),
              np.full((4, 4), 0.25, np.float32))
_WEXP = np.zeros((NUM_HEADS, MSG_W), np.float32)
for _h in range(NUM_HEADS):
    for _k in range(K):
        for _c in range(ATTN_VALUE_CH):
            _WEXP[_h, _k * VALUE_TOT + _h * ATTN_VALUE_CH + _c] = 1.0


def _scaled_silu(x):
    return x * jax.nn.sigmoid(x) * SILU_SCALE


def _smooth_leaky_relu(x, alpha=0.2):
    return ((1.0 + alpha) / 2.0) * x + ((1.0 - alpha) / 2.0) * x * (2.0 * jax.nn.sigmoid(x) - 1.0)


def _ln_mm(x, mmat, g, b, eps=1e-5):
    mu = jnp.dot(x, mmat, preferred_element_type=jnp.float32)
    d = x - mu
    var = jnp.dot(d * d, mmat, preferred_element_type=jnp.float32)
    return d * jax.lax.rsqrt(var + eps) * g + b


def _fused_edge_kernel(
        idx_ref, ed_ref, wig_ref, wiginv_ref, ntab_ref,
        rw1_ref, rb1_ref, rg1_ref, rbe1_ref,
        rw2_ref, rb2_ref, rg2_ref, rbe2_ref,
        rw3_ref, rb3_ref,
        wcat1_ref, bcat1_ref, g1_ref,
        tgx_ref, fgx_ref,
        wcat2_ref, bcat2_ref, g2_ref,
        m16_ref, m8_ref, ag8_ref, ab8_ref, adot_ref, wexp_ref,
        prot_ref, trot_ref, pinv_ref, tinv_ref,
        out_ref, src_rows, tgt_rows):
    f32 = jnp.float32
    te = EDGE_TILE

    for mi in range(te):
        src_rows[mi] = ntab_ref[idx_ref[0, 0, mi], 0]
        tgt_rows[mi] = ntab_ref[idx_ref[0, 1, mi], 0]
    xs = src_rows[...]
    xt = tgt_rows[...]

    m16 = m16_ref[...]
    x_edge = jnp.concatenate(
        [ed_ref[...], xs[:, 72:88], xt[:, 88:104]], axis=1)
    h = jnp.dot(x_edge, rw1_ref[...], preferred_element_type=f32) + rb1_ref[...]
    h = _scaled_silu(_ln_mm(h, m16, rg1_ref[...], rbe1_ref[...]))
    h = jnp.dot(h, rw2_ref[...], preferred_element_type=f32) + rb2_ref[...]
    h = _scaled_silu(_ln_mm(h, m16, rg2_ref[...], rbe2_ref[...]))
    rad_full = jnp.dot(h, rw3_ref[...], preferred_element_type=f32) + rb3_ref[...]

    X = jnp.concatenate([xs[:, :72], xt[:, :72]], axis=1)
    wigi = wiginv_ref[...]
    rot_cat = None
    for j in range(K):
        d = jnp.dot(wigi, prot_ref[j], preferred_element_type=f32)
        v = jnp.dot(X, trot_ref[j], preferred_element_type=f32)
        rot_cat = d * v if rot_cat is None else rot_cat + d * v

    xr = rot_cat * rad_full
    y1 = jnp.dot(xr, wcat1_ref[...], preferred_element_type=f32) + bcat1_ref[...]
    alpha_feat = y1[:, :ALPHA_TOT]
    gating = y1[:, ALPHA_TOT:EXTRA_M0]
    hid_cat = jnp.dot(y1, g1_ref[...], preferred_element_type=f32)

    g = _scaled_silu(jnp.dot(hid_cat, tgx_ref[...], preferred_element_type=f32))
    y = jnp.dot(g, fgx_ref[...], preferred_element_type=f32)
    act_cat = jnp.concatenate(
        [_scaled_silu(gating), y[:, HIDDEN_CH:]], axis=1)

    y2 = jnp.dot(act_cat, wcat2_ref[...], preferred_element_type=f32) + bcat2_ref[...]
    vcat = jnp.dot(y2, g2_ref[...], preferred_element_type=f32)

    ln = _ln_mm(alpha_feat, m8_ref[...], ag8_ref[...], ab8_ref[...])
    alpha = jnp.dot(_smooth_leaky_relu(ln), adot_ref[...],
                    preferred_element_type=f32)

    wigf = wig_ref[...]
    msg = None
    for j in range(K):
        d = jnp.dot(wigf, pinv_ref[j], preferred_element_type=f32)
        v = jnp.dot(vcat, tinv_ref[j], preferred_element_type=f32)
        msg = d * v if msg is None else msg + d * v

    w = jnp.exp(alpha)
    wfull = jnp.dot(w, wexp_ref[...], preferred_element_type=f32)

    out_ref[...] = jnp.concatenate(
        [msg * wfull, w, jnp.zeros((te, PAD_W), f32)], axis=-1)


def _fused_edge_messages(idx2, edge_distance, wig2, wiginv2, node_tab, weights):
    E = wig2.shape[0]
    te = EDGE_TILE
    ns = E // te // 2
    grid = (2, ns)

    def row_spec(width):
        return pl.BlockSpec((te, width), lambda c, i: (c * ns + i, 0))

    in_specs = [
        pl.BlockSpec((1, 2, te), lambda c, i: (c * ns + i, 0, 0),
                     memory_space=pltpu.SMEM),
        row_spec(edge_distance.shape[1]),
        row_spec(wig2.shape[1]), row_spec(wiginv2.shape[1]),
        pl.BlockSpec(node_tab.shape, lambda c, i: (0, 0, 0)),
    ]
    in_specs += [pl.BlockSpec(w.shape, lambda c, i, n=w.ndim: (0,) * n)
                 for w in weights]

    out_shape = jax.ShapeDtypeStruct((E, PACK_W), jnp.float32)
    out_specs = pl.BlockSpec((te, PACK_W), lambda c, i: (c * ns + i, 0))

    return pl.pallas_call(
        _fused_edge_kernel,
        out_shape=out_shape,
        grid=grid,
        in_specs=in_specs,
        out_specs=out_specs,
        scratch_shapes=[pltpu.VMEM((te, NODE_W), jnp.float32),
                        pltpu.VMEM((te, NODE_W), jnp.float32)],
        compiler_params=pltpu.CompilerParams(
            dimension_semantics=("parallel", "arbitrary"),
            vmem_limit_bytes=96 * 1024 * 1024),
    )(idx2, edge_distance, wig2, wiginv2, node_tab, *weights)


def _node_proj_kernel(acc_ref, w_ref, b_ref, wexp_ref, o_ref):
    acc = acc_ref[...]
    x = acc[:, :MSG_W]
    z = acc[:, MSG_W:MSG_W + NUM_HEADS]
    inv = 1.0 / (z + 1e-16)
    inv_full = jnp.dot(inv, wexp_ref[...], preferred_element_type=jnp.float32)
    o_ref[...] = jnp.dot(x * inv_full, w_ref[...],
                         preferred_element_type=jnp.float32) + b_ref[...]


def _node_divide_project(acc, wbd_pad, bias_row, wexp):
    N = acc.shape[0]
    return pl.pallas_call(
        _node_proj_kernel,
        out_shape=jax.ShapeDtypeStruct((N, PROJ_PACK_W), jnp.float32),
        grid=(1,),
        in_specs=[pl.BlockSpec((N, PACK_W), lambda i: (0, 0)),
                  pl.BlockSpec((MSG_W, PROJ_PACK_W), lambda i: (0, 0)),
                  pl.BlockSpec((1, PROJ_PACK_W), lambda i: (0, 0)),
                  pl.BlockSpec(wexp.shape, lambda i: (0, 0))],
        out_specs=pl.BlockSpec((N, PROJ_PACK_W), lambda i: (0, 0)),
    )(acc, wbd_pad, bias_row, wexp)


def kernel(x_emb, atomic_numbers, edge_distance, edge_index, wigner, wigner_inv,
           to_grid, from_grid, source_embedding, target_embedding,
           rad1_w1, rad1_b1, rad1_ln1_g, rad1_ln1_b, rad1_w2, rad1_b2,
           rad1_ln2_g, rad1_ln2_b, rad1_w3, rad1_b3,
           conv1_w0, conv1_b0, conv1_w1, conv1_w2,
           conv2_w0, conv2_b0, conv2_w1, conv2_w2,
           alpha_ln_g, alpha_ln_b, alpha_dot, proj_w, proj_b):
    E = edge_index.shape[1]
    N = x_emb.shape[0]
    te = EDGE_TILE
    src, tgt = edge_index[0], edge_index[1]

    node_tab = jnp.concatenate(
        [x_emb.reshape(N, K * SPHERE_CH),
         source_embedding[atomic_numbers],
         target_embedding[atomic_numbers],
         jnp.zeros((N, NODE_W - K * SPHERE_CH - 32), jnp.float32)],
        axis=1).reshape(N, 1, NODE_W)

    idx2 = jnp.stack([src, tgt], axis=0).reshape(2, E // te, te).transpose(1, 0, 2)

    wig2 = wigner.reshape(E, K * K)
    wiginv2 = wigner_inv.reshape(E, K * K)

    eye_h = jnp.eye(HIDDEN_CH, dtype=jnp.float32)
    tg_exp = jnp.kron(to_grid.T, eye_h)
    fg_exp = jnp.kron(from_grid, eye_h)

    radx = jnp.asarray(_RADX)
    rw3f = jnp.dot(rad1_w3, radx)
    rb3f = jnp.dot(rad1_b3.reshape(1, -1), radx)
    wcat1 = jnp.concatenate(
        [jnp.asarray(_sel_t(IDX0, C_IN1)) @ conv1_w0,
         jnp.asarray(_sel_t(PLUS1, C_IN1)) @ conv1_w1,
         jnp.asarray(_sel_t(MINUS1, C_IN1)) @ conv1_w1,
         jnp.asarray(_sel_t(PLUS2, C_IN1)) @ conv1_w2,
         jnp.asarray(_sel_t(MINUS2, C_IN1)) @ conv1_w2], axis=1)
    bcat1 = jnp.concatenate(
        [conv1_b0.reshape(1, -1), jnp.zeros((1, 96), jnp.float32)], axis=1)
    wcat2 = jnp.concatenate(
        [jnp.asarray(_sel_t(IDX0, HIDDEN_CH)) @ conv2_w0,
         jnp.asarray(_sel_t(PLUS1, HIDDEN_CH)) @ conv2_w1,
         jnp.asarray(_sel_t(MINUS1, HIDDEN_CH)) @ conv2_w1,
         jnp.asarray(_sel_t(PLUS2, HIDDEN_CH)) @ conv2_w2,
         jnp.asarray(_sel_t(MINUS2, HIDDEN_CH)) @ conv2_w2], axis=1)
    bcat2 = jnp.concatenate(
        [conv2_b0.reshape(1, -1), jnp.zeros((1, 96), jnp.float32)], axis=1)

    ag8 = jnp.tile(alpha_ln_g, 2).reshape(1, ALPHA_TOT)
    ab8 = jnp.tile(alpha_ln_b, 2).reshape(1, ALPHA_TOT)
    adot_bd = jnp.zeros((ALPHA_TOT, NUM_HEADS), jnp.float32)
    adot_bd = adot_bd.at[:ATTN_ALPHA_CH, 0].set(alpha_dot[0])
    adot_bd = adot_bd.at[ATTN_ALPHA_CH:, 1].set(alpha_dot[1])

    weights = [
        rad1_w1, rad1_b1.reshape(1, -1),
        rad1_ln1_g.reshape(1, -1), rad1_ln1_b.reshape(1, -1),
        rad1_w2, rad1_b2.reshape(1, -1),
        rad1_ln2_g.reshape(1, -1), rad1_ln2_b.reshape(1, -1),
        rw3f, rb3f,
        wcat1, bcat1, jnp.asarray(_G1),
        tg_exp, fg_exp,
        wcat2, bcat2, jnp.asarray(_G2),
        jnp.asarray(_M16), jnp.asarray(_M8), ag8, ab8, adot_bd,
        jnp.asarray(_WEXP),
        jnp.asarray(_PROT), jnp.asarray(_TROT),
        jnp.asarray(_PINV), jnp.asarray(_TINV),
    ]

    packed = _fused_edge_messages(idx2, edge_distance, wig2, wiginv2,
                                  node_tab, weights)

    acc = jax.ops.segment_sum(packed, tgt, num_segments=N)

    w_per = jnp.transpose(proj_w[L_PER_COEF], (0, 2, 1))
    eye_k = jnp.eye(K, dtype=jnp.float32)
    wbd = (eye_k[:, None, :, None] * w_per[:, :, None, :]).reshape(MSG_W, PROJ_W)
    wbd_pad = jnp.zeros((MSG_W, PROJ_PACK_W), jnp.float32).at[:, :PROJ_W].set(wbd)
    bias_row = jnp.zeros((1, PROJ_PACK_W), jnp.float32).at[0, :OUTPUT_CH].set(proj_b)
    out = _node_divide_project(acc, wbd_pad, bias_row, jnp.asarray(_WEXP))[:, :PROJ_W]
    return out.reshape(N, K, OUTPUT_CH)


# speedup vs baseline: 1.8131x; 1.1311x over previous
import math
import numpy as np

import jax
import jax.numpy as jnp
from jax.experimental import pallas as pl
from jax.experimental.pallas import tpu as pltpu

LMAX = 2
MMAX = 2
K = (LMAX + 1) ** 2
SPHERE_CH = 8
HIDDEN_CH = 8
NUM_HEADS = 2
ATTN_ALPHA_CH = 4
ATTN_VALUE_CH = 4
OUTPUT_CH = 8
SILU_SCALE = 1.0 / 0.6

C_IN1 = 2 * SPHERE_CH
ALPHA_TOT = NUM_HEADS * ATTN_ALPHA_CH
VALUE_TOT = NUM_HEADS * ATTN_VALUE_CH
EXTRA_M0 = ALPHA_TOT + HIDDEN_CH

EDGE_TILE = 512

MSG_W = K * VALUE_TOT
PACK_W = 128
PAD_W = PACK_W - MSG_W - NUM_HEADS

PROJ_W = K * OUTPUT_CH
PROJ_PACK_W = 128

NODE_W = 128
X_W = K * C_IN1

IDX0 = [0, 2, 6]
PLUS1, MINUS1 = [3, 7], [1, 5]
PLUS2, MINUS2 = [8], [4]

L_PER_COEF = np.concatenate([[l] * (2 * l + 1) for l in range(LMAX + 1)]).astype(np.int32)


def _expansion_mats():
    prot = np.zeros((K, K * K, X_W), np.float32)
    trot = np.zeros((K, X_W, X_W), np.float32)
    pinv = np.zeros((K, K * K, MSG_W), np.float32)
    tinv = np.zeros((K, MSG_W, MSG_W), np.float32)
    for j in range(K):
        for k in range(K):
            prot[j, j * K + k, k * C_IN1:(k + 1) * C_IN1] = 1.0
            pinv[j, j * K + k, k * VALUE_TOT:(k + 1) * VALUE_TOT] = 1.0
            for c in range(SPHERE_CH):
                trot[j, j * SPHERE_CH + c, k * C_IN1 + c] = 1.0
                trot[j, K * SPHERE_CH + j * SPHERE_CH + c,
                     k * C_IN1 + SPHERE_CH + c] = 1.0
            for c in range(VALUE_TOT):
                tinv[j, j * VALUE_TOT + c, k * VALUE_TOT + c] = 1.0
    return prot, trot, pinv, tinv


def _sel_t(coefs, ch):
    s = np.zeros((K * ch, len(coefs) * ch), np.float32)
    for t, i in enumerate(coefs):
        for c in range(ch):
            s[i * ch + c, t * ch + c] = 1.0
    return s


def _gather_mat(ch, extra):
    n0, n1, n2 = len(IDX0) * ch, len(PLUS1) * ch, len(PLUS2) * ch
    cols = extra + n0 + 2 * (2 * n1) + 2 * (2 * n2)
    g = np.zeros((cols, K * ch), np.float32)
    o = extra
    for t, i in enumerate(IDX0):
        for c in range(ch):
            g[o + t * ch + c, i * ch + c] = 1.0
    o += n0
    for t, i in enumerate(PLUS1):
        for c in range(ch):
            g[o + t * ch + c, i * ch + c] = 1.0
    o += n1
    for t, i in enumerate(MINUS1):
        for c in range(ch):
            g[o + t * ch + c, i * ch + c] = 1.0
    o += n1
    for t, i in enumerate(MINUS1):
        for c in range(ch):
            g[o + t * ch + c, i * ch + c] = 1.0
    o += n1
    for t, i in enumerate(PLUS1):
        for c in range(ch):
            g[o + t * ch + c, i * ch + c] = -1.0
    o += n1
    for t, i in enumerate(PLUS2):
        for c in range(ch):
            g[o + t * ch + c, i * ch + c] = 1.0
    o += n2
    for t, i in enumerate(MINUS2):
        for c in range(ch):
            g[o + t * ch + c, i * ch + c] = 1.0
    o += n2
    for t, i in enumerate(MINUS2):
        for c in range(ch):
            g[o + t * ch + c, i * ch + c] = 1.0
    o += n2
    for t, i in enumerate(PLUS2):
        for c in range(ch):
            g[o + t * ch + c, i * ch + c] = -1.0
    return g


def _rad_expand():
    r = np.zeros((96, X_W), np.float32)
    for t, i in enumerate(IDX0):
        for c in range(C_IN1):
            r[t * C_IN1 + c, i * C_IN1 + c] = 1.0
    for t in range(len(PLUS1)):
        for c in range(C_IN1):
            r[48 + t * C_IN1 + c, PLUS1[t] * C_IN1 + c] = 1.0
            r[48 + t * C_IN1 + c, MINUS1[t] * C_IN1 + c] = 1.0
    for c in range(C_IN1):
        r[80 + c, PLUS2[0] * C_IN1 + c] = 1.0
        r[80 + c, MINUS2[0] * C_IN1 + c] = 1.0
    return r


_PROT, _TROT, _PINV, _TINV = _expansion_mats()
_G1 = _gather_mat(HIDDEN_CH, EXTRA_M0)
_G2 = _gather_mat(VALUE_TOT, 0)
_RADX = _rad_expand()
_M16 = np.full((16, 16), 1.0 / 16.0, np.float32)
_M8 = np.kron(np.eye(2, dtype=np.float32),
              np.full((4, 4), 0.25, np.float32))
_WEXP = np.zeros((NUM_HEADS, MSG_W), np.float32)
for _h in range(NUM_HEADS):
    for _k in range(K):
        for _c in range(ATTN_VALUE_CH):
            _WEXP[_h, _k * VALUE_TOT + _h * ATTN_VALUE_CH + _c] = 1.0


def _scaled_silu(x):
    return x * jax.nn.sigmoid(x) * SILU_SCALE


def _smooth_leaky_relu(x, alpha=0.2):
    return ((1.0 + alpha) / 2.0) * x + ((1.0 - alpha) / 2.0) * x * (2.0 * jax.nn.sigmoid(x) - 1.0)


def _ln_mm(x, mmat, g, b, eps=1e-5):
    mu = jnp.dot(x, mmat, preferred_element_type=jnp.float32)
    d = x - mu
    var = jnp.dot(d * d, mmat, preferred_element_type=jnp.float32)
    return d * jax.lax.rsqrt(var + eps) * g + b


def _fused_edge_kernel(
        idx_ref, ed_ref, wig_ref, wiginv_ref, ntab_ref,
        rw1_ref, rb1_ref, rg1_ref, rbe1_ref,
        rw2_ref, rb2_ref, rg2_ref, rbe2_ref,
        rw3_ref, rb3_ref,
        wcat1_ref, bcat1_ref, g1_ref,
        tgx_ref, fgx_ref,
        wcat2_ref, bcat2_ref, g2_ref,
        m16_ref, m8_ref, ag8_ref, ab8_ref, adot_ref, wexp_ref,
        prot_ref, trot_ref, pinv_ref, tinv_ref,
        out_ref, src_rows, tgt_rows, pk, acc0, acc1, acc2, acc3):
    f32 = jnp.float32
    te = EDGE_TILE
    accs = [acc0, acc1, acc2, acc3]

    @pl.when(pl.program_id(1) == 0)
    def _init():
        for a in accs:
            a[...] = jnp.zeros_like(a)

    for mi in range(te):
        src_rows[mi] = ntab_ref[idx_ref[0, 0, mi], 0]
        tgt_rows[mi] = ntab_ref[idx_ref[0, 1, mi], 0]
    xs = src_rows[...]
    xt = tgt_rows[...]

    m16 = m16_ref[...]
    x_edge = jnp.concatenate(
        [ed_ref[...], xs[:, 72:88], xt[:, 88:104]], axis=1)
    h = jnp.dot(x_edge, rw1_ref[...], preferred_element_type=f32) + rb1_ref[...]
    h = _scaled_silu(_ln_mm(h, m16, rg1_ref[...], rbe1_ref[...]))
    h = jnp.dot(h, rw2_ref[...], preferred_element_type=f32) + rb2_ref[...]
    h = _scaled_silu(_ln_mm(h, m16, rg2_ref[...], rbe2_ref[...]))
    rad_full = jnp.dot(h, rw3_ref[...], preferred_element_type=f32) + rb3_ref[...]

    X = jnp.concatenate([xs[:, :72], xt[:, :72]], axis=1)
    wigi = wiginv_ref[...]
    rot_cat = None
    for j in range(K):
        d = jnp.dot(wigi, prot_ref[j], preferred_element_type=f32)
        v = jnp.dot(X, trot_ref[j], preferred_element_type=f32)
        rot_cat = d * v if rot_cat is None else rot_cat + d * v

    xr = rot_cat * rad_full
    y1 = jnp.dot(xr, wcat1_ref[...], preferred_element_type=f32) + bcat1_ref[...]
    alpha_feat = y1[:, :ALPHA_TOT]
    gating = y1[:, ALPHA_TOT:EXTRA_M0]
    hid_cat = jnp.dot(y1, g1_ref[...], preferred_element_type=f32)

    g = _scaled_silu(jnp.dot(hid_cat, tgx_ref[...], preferred_element_type=f32))
    y = jnp.dot(g, fgx_ref[...], preferred_element_type=f32)
    act_cat = jnp.concatenate(
        [_scaled_silu(gating), y[:, HIDDEN_CH:]], axis=1)

    y2 = jnp.dot(act_cat, wcat2_ref[...], preferred_element_type=f32) + bcat2_ref[...]
    vcat = jnp.dot(y2, g2_ref[...], preferred_element_type=f32)

    ln = _ln_mm(alpha_feat, m8_ref[...], ag8_ref[...], ab8_ref[...])
    alpha = jnp.dot(_smooth_leaky_relu(ln), adot_ref[...],
                    preferred_element_type=f32)

    wigf = wig_ref[...]
    msg = None
    for j in range(K):
        d = jnp.dot(wigf, pinv_ref[j], preferred_element_type=f32)
        v = jnp.dot(vcat, tinv_ref[j], preferred_element_type=f32)
        msg = d * v if msg is None else msg + d * v

    w = jnp.exp(alpha)
    wfull = jnp.dot(w, wexp_ref[...], preferred_element_type=f32)

    pk[...] = jnp.concatenate(
        [msg * wfull, w, jnp.zeros((te, PAD_W), f32)], axis=-1)

    for mi in range(te):
        a = accs[mi % 4]
        t = idx_ref[0, 1, mi]
        a[t, 0] = a[t, 0] + pk[mi]

    @pl.when(pl.program_id(1) == pl.num_programs(1) - 1)
    def _flush():
        out_ref[0] = ((acc0[...] + acc1[...]) + (acc2[...] + acc3[...]))


def _fused_edge_messages(idx2, edge_distance, wig2, wiginv2, node_tab, weights):
    E = wig2.shape[0]
    N = node_tab.shape[0]
    te = EDGE_TILE
    ns = E // te // 2
    grid = (2, ns)

    def row_spec(width):
        return pl.BlockSpec((te, width), lambda c, i: (c * ns + i, 0))

    in_specs = [
        pl.BlockSpec((1, 2, te), lambda c, i: (c * ns + i, 0, 0),
                     memory_space=pltpu.SMEM),
        row_spec(edge_distance.shape[1]),
        row_spec(wig2.shape[1]), row_spec(wiginv2.shape[1]),
        pl.BlockSpec(node_tab.shape, lambda c, i: (0, 0, 0)),
    ]
    in_specs += [pl.BlockSpec(w.shape, lambda c, i, n=w.ndim: (0,) * n)
                 for w in weights]

    out_shape = jax.ShapeDtypeStruct((2, N, 1, PACK_W), jnp.float32)
    out_specs = pl.BlockSpec((1, N, 1, PACK_W), lambda c, i: (c, 0, 0, 0))

    return pl.pallas_call(
        _fused_edge_kernel,
        out_shape=out_shape,
        grid=grid,
        in_specs=in_specs,
        out_specs=out_specs,
        scratch_shapes=[pltpu.VMEM((te, NODE_W), jnp.float32),
                        pltpu.VMEM((te, NODE_W), jnp.float32),
                        pltpu.VMEM((te, PACK_W), jnp.float32),
                        pltpu.VMEM((N, 1, PACK_W), jnp.float32),
                        pltpu.VMEM((N, 1, PACK_W), jnp.float32),
                        pltpu.VMEM((N, 1, PACK_W), jnp.float32),
                        pltpu.VMEM((N, 1, PACK_W), jnp.float32)],
        compiler_params=pltpu.CompilerParams(
            dimension_semantics=("parallel", "arbitrary"),
            vmem_limit_bytes=96 * 1024 * 1024),
    )(idx2, edge_distance, wig2, wiginv2, node_tab, *weights)


def _node_proj_kernel(acc_ref, w_ref, b_ref, wexp_ref, o_ref):
    acc = acc_ref[...]
    x = acc[:, :MSG_W]
    z = acc[:, MSG_W:MSG_W + NUM_HEADS]
    inv = 1.0 / (z + 1e-16)
    inv_full = jnp.dot(inv, wexp_ref[...], preferred_element_type=jnp.float32)
    o_ref[...] = jnp.dot(x * inv_full, w_ref[...],
                         preferred_element_type=jnp.float32) + b_ref[...]


def _node_divide_project(acc, wbd_pad, bias_row, wexp):
    N = acc.shape[0]
    return pl.pallas_call(
        _node_proj_kernel,
        out_shape=jax.ShapeDtypeStruct((N, PROJ_PACK_W), jnp.float32),
        grid=(1,),
        in_specs=[pl.BlockSpec((N, PACK_W), lambda i: (0, 0)),
                  pl.BlockSpec((MSG_W, PROJ_PACK_W), lambda i: (0, 0)),
                  pl.BlockSpec((1, PROJ_PACK_W), lambda i: (0, 0)),
                  pl.BlockSpec(wexp.shape, lambda i: (0, 0))],
        out_specs=pl.BlockSpec((N, PROJ_PACK_W), lambda i: (0, 0)),
    )(acc, wbd_pad, bias_row, wexp)


def kernel(x_emb, atomic_numbers, edge_distance, edge_index, wigner, wigner_inv,
           to_grid, from_grid, source_embedding, target_embedding,
           rad1_w1, rad1_b1, rad1_ln1_g, rad1_ln1_b, rad1_w2, rad1_b2,
           rad1_ln2_g, rad1_ln2_b, rad1_w3, rad1_b3,
           conv1_w0, conv1_b0, conv1_w1, conv1_w2,
           conv2_w0, conv2_b0, conv2_w1, conv2_w2,
           alpha_ln_g, alpha_ln_b, alpha_dot, proj_w, proj_b):
    E = edge_index.shape[1]
    N = x_emb.shape[0]
    te = EDGE_TILE
    src, tgt = edge_index[0], edge_index[1]

    node_tab = jnp.concatenate(
        [x_emb.reshape(N, K * SPHERE_CH),
         source_embedding[atomic_numbers],
         target_embedding[atomic_numbers],
         jnp.zeros((N, NODE_W - K * SPHERE_CH - 32), jnp.float32)],
        axis=1).reshape(N, 1, NODE_W)

    idx2 = jnp.stack([src, tgt], axis=0).reshape(2, E // te, te).transpose(1, 0, 2)

    wig2 = wigner.reshape(E, K * K)
    wiginv2 = wigner_inv.reshape(E, K * K)

    eye_h = jnp.eye(HIDDEN_CH, dtype=jnp.float32)
    tg_exp = jnp.kron(to_grid.T, eye_h)
    fg_exp = jnp.kron(from_grid, eye_h)

    radx = jnp.asarray(_RADX)
    rw3f = jnp.dot(rad1_w3, radx)
    rb3f = jnp.dot(rad1_b3.reshape(1, -1), radx)
    wcat1 = jnp.concatenate(
        [jnp.asarray(_sel_t(IDX0, C_IN1)) @ conv1_w0,
         jnp.asarray(_sel_t(PLUS1, C_IN1)) @ conv1_w1,
         jnp.asarray(_sel_t(MINUS1, C_IN1)) @ conv1_w1,
         jnp.asarray(_sel_t(PLUS2, C_IN1)) @ conv1_w2,
         jnp.asarray(_sel_t(MINUS2, C_IN1)) @ conv1_w2], axis=1)
    bcat1 = jnp.concatenate(
        [conv1_b0.reshape(1, -1), jnp.zeros((1, 96), jnp.float32)], axis=1)
    wcat2 = jnp.concatenate(
        [jnp.asarray(_sel_t(IDX0, HIDDEN_CH)) @ conv2_w0,
         jnp.asarray(_sel_t(PLUS1, HIDDEN_CH)) @ conv2_w1,
         jnp.asarray(_sel_t(MINUS1, HIDDEN_CH)) @ conv2_w1,
         jnp.asarray(_sel_t(PLUS2, HIDDEN_CH)) @ conv2_w2,
         jnp.asarray(_sel_t(MINUS2, HIDDEN_CH)) @ conv2_w2], axis=1)
    bcat2 = jnp.concatenate(
        [conv2_b0.reshape(1, -1), jnp.zeros((1, 96), jnp.float32)], axis=1)

    ag8 = jnp.tile(alpha_ln_g, 2).reshape(1, ALPHA_TOT)
    ab8 = jnp.tile(alpha_ln_b, 2).reshape(1, ALPHA_TOT)
    adot_bd = jnp.zeros((ALPHA_TOT, NUM_HEADS), jnp.float32)
    adot_bd = adot_bd.at[:ATTN_ALPHA_CH, 0].set(alpha_dot[0])
    adot_bd = adot_bd.at[ATTN_ALPHA_CH:, 1].set(alpha_dot[1])

    weights = [
        rad1_w1, rad1_b1.reshape(1, -1),
        rad1_ln1_g.reshape(1, -1), rad1_ln1_b.reshape(1, -1),
        rad1_w2, rad1_b2.reshape(1, -1),
        rad1_ln2_g.reshape(1, -1), rad1_ln2_b.reshape(1, -1),
        rw3f, rb3f,
        wcat1, bcat1, jnp.asarray(_G1),
        tg_exp, fg_exp,
        wcat2, bcat2, jnp.asarray(_G2),
        jnp.asarray(_M16), jnp.asarray(_M8), ag8, ab8, adot_bd,
        jnp.asarray(_WEXP),
        jnp.asarray(_PROT), jnp.asarray(_TROT),
        jnp.asarray(_PINV), jnp.asarray(_TINV),
    ]

    parts = _fused_edge_messages(idx2, edge_distance, wig2, wiginv2,
                                 node_tab, weights)
    acc = (parts[0] + parts[1]).reshape(N, PACK_W)

    w_per = jnp.transpose(proj_w[L_PER_COEF], (0, 2, 1))
    eye_k = jnp.eye(K, dtype=jnp.float32)
    wbd = (eye_k[:, None, :, None] * w_per[:, :, None, :]).reshape(MSG_W, PROJ_W)
    wbd_pad = jnp.zeros((MSG_W, PROJ_PACK_W), jnp.float32).at[:, :PROJ_W].set(wbd)
    bias_row = jnp.zeros((1, PROJ_PACK_W), jnp.float32).at[0, :OUTPUT_CH].set(proj_b)
    out = _node_divide_project(acc, wbd_pad, bias_row, jnp.asarray(_WEXP))[:, :PROJ_W]
    return out.reshape(N, K, OUTPUT_CH)


# speedup vs baseline: 2.0763x; 1.1451x over previous
import math
import numpy as np

import jax
import jax.numpy as jnp
from jax.experimental import pallas as pl
from jax.experimental.pallas import tpu as pltpu

LMAX = 2
MMAX = 2
K = (LMAX + 1) ** 2
SPHERE_CH = 8
HIDDEN_CH = 8
NUM_HEADS = 2
ATTN_ALPHA_CH = 4
ATTN_VALUE_CH = 4
OUTPUT_CH = 8
SILU_SCALE = 1.0 / 0.6

C_IN1 = 2 * SPHERE_CH
ALPHA_TOT = NUM_HEADS * ATTN_ALPHA_CH
VALUE_TOT = NUM_HEADS * ATTN_VALUE_CH
EXTRA_M0 = ALPHA_TOT + HIDDEN_CH

EDGE_TILE = 1024

MSG_W = K * VALUE_TOT
PACK_W = 128
PAD_W = PACK_W - MSG_W - NUM_HEADS

PROJ_W = K * OUTPUT_CH
PROJ_PACK_W = 128

NODE_W = 128
X_W = K * C_IN1

IDX0 = [0, 2, 6]
PLUS1, MINUS1 = [3, 7], [1, 5]
PLUS2, MINUS2 = [8], [4]

L_PER_COEF = np.concatenate([[l] * (2 * l + 1) for l in range(LMAX + 1)]).astype(np.int32)


def _expansion_mats():
    prot = np.zeros((K, K * K, X_W), np.float32)
    trot = np.zeros((K, X_W, X_W), np.float32)
    pinv = np.zeros((K, K * K, MSG_W), np.float32)
    tinv = np.zeros((K, MSG_W, MSG_W), np.float32)
    for j in range(K):
        for k in range(K):
            prot[j, j * K + k, k * C_IN1:(k + 1) * C_IN1] = 1.0
            pinv[j, j * K + k, k * VALUE_TOT:(k + 1) * VALUE_TOT] = 1.0
            for c in range(SPHERE_CH):
                trot[j, j * SPHERE_CH + c, k * C_IN1 + c] = 1.0
                trot[j, K * SPHERE_CH + j * SPHERE_CH + c,
                     k * C_IN1 + SPHERE_CH + c] = 1.0
            for c in range(VALUE_TOT):
                tinv[j, j * VALUE_TOT + c, k * VALUE_TOT + c] = 1.0
    return prot, trot, pinv, tinv


def _sel_t(coefs, ch):
    s = np.zeros((K * ch, len(coefs) * ch), np.float32)
    for t, i in enumerate(coefs):
        for c in range(ch):
            s[i * ch + c, t * ch + c] = 1.0
    return s


def _gather_mat(ch, extra):
    n0, n1, n2 = len(IDX0) * ch, len(PLUS1) * ch, len(PLUS2) * ch
    cols = extra + n0 + 2 * (2 * n1) + 2 * (2 * n2)
    g = np.zeros((cols, K * ch), np.float32)
    o = extra
    for t, i in enumerate(IDX0):
        for c in range(ch):
            g[o + t * ch + c, i * ch + c] = 1.0
    o += n0
    for t, i in enumerate(PLUS1):
        for c in range(ch):
            g[o + t * ch + c, i * ch + c] = 1.0
    o += n1
    for t, i in enumerate(MINUS1):
        for c in range(ch):
            g[o + t * ch + c, i * ch + c] = 1.0
    o += n1
    for t, i in enumerate(MINUS1):
        for c in range(ch):
            g[o + t * ch + c, i * ch + c] = 1.0
    o += n1
    for t, i in enumerate(PLUS1):
        for c in range(ch):
            g[o + t * ch + c, i * ch + c] = -1.0
    o += n1
    for t, i in enumerate(PLUS2):
        for c in range(ch):
            g[o + t * ch + c, i * ch + c] = 1.0
    o += n2
    for t, i in enumerate(MINUS2):
        for c in range(ch):
            g[o + t * ch + c, i * ch + c] = 1.0
    o += n2
    for t, i in enumerate(MINUS2):
        for c in range(ch):
            g[o + t * ch + c, i * ch + c] = 1.0
    o += n2
    for t, i in enumerate(PLUS2):
        for c in range(ch):
            g[o + t * ch + c, i * ch + c] = -1.0
    return g


def _rad_expand():
    r = np.zeros((96, X_W), np.float32)
    for t, i in enumerate(IDX0):
        for c in range(C_IN1):
            r[t * C_IN1 + c, i * C_IN1 + c] = 1.0
    for t in range(len(PLUS1)):
        for c in range(C_IN1):
            r[48 + t * C_IN1 + c, PLUS1[t] * C_IN1 + c] = 1.0
            r[48 + t * C_IN1 + c, MINUS1[t] * C_IN1 + c] = 1.0
    for c in range(C_IN1):
        r[80 + c, PLUS2[0] * C_IN1 + c] = 1.0
        r[80 + c, MINUS2[0] * C_IN1 + c] = 1.0
    return r


_PROT, _TROT, _PINV, _TINV = _expansion_mats()
_G1 = _gather_mat(HIDDEN_CH, EXTRA_M0)
_G2 = _gather_mat(VALUE_TOT, 0)
_RADX = _rad_expand()
_M16 = np.full((16, 16), 1.0 / 16.0, np.float32)
_M8 = np.kron(np.eye(2, dtype=np.float32),
              np.full((4, 4), 0.25, np.float32))
_WEXP = np.zeros((NUM_HEADS, MSG_W), np.float32)
for _h in range(NUM_HEADS):
    for _k in range(K):
        for _c in range(ATTN_VALUE_CH):
            _WEXP[_h, _k * VALUE_TOT + _h * ATTN_VALUE_CH + _c] = 1.0


def _scaled_silu(x):
    return x * jax.nn.sigmoid(x) * SILU_SCALE


def _smooth_leaky_relu(x, alpha=0.2):
    return ((1.0 + alpha) / 2.0) * x + ((1.0 - alpha) / 2.0) * x * (2.0 * jax.nn.sigmoid(x) - 1.0)


def _ln_mm(x, mmat, g, b, eps=1e-5):
    mu = jnp.dot(x, mmat, preferred_element_type=jnp.float32)
    d = x - mu
    var = jnp.dot(d * d, mmat, preferred_element_type=jnp.float32)
    return d * jax.lax.rsqrt(var + eps) * g + b


def _fused_edge_kernel(
        idx_ref, ed_ref, wig_ref, wiginv_ref, ntab_ref,
        rw1_ref, rb1_ref, rg1_ref, rbe1_ref,
        rw2_ref, rb2_ref, rg2_ref, rbe2_ref,
        rw3_ref, rb3_ref,
        wcat1_ref, bcat1_ref, g1_ref,
        tgx_ref, fgx_ref,
        wcat2_ref, bcat2_ref, g2_ref,
        m16_ref, m8_ref, ag8_ref, ab8_ref, adot_ref, wexp_ref,
        prot_ref, trot_ref, pinv_ref, tinv_ref,
        out_ref, src_rows, tgt_rows, pk, acc0, acc1, acc2, acc3):
    f32 = jnp.float32
    te = EDGE_TILE
    accs = [acc0, acc1, acc2, acc3]

    @pl.when(pl.program_id(1) == 0)
    def _init():
        for a in accs:
            a[...] = jnp.zeros_like(a)

    for mi in range(te):
        src_rows[mi] = ntab_ref[idx_ref[0, 0, mi], 0]
        tgt_rows[mi] = ntab_ref[idx_ref[0, 1, mi], 0]
    xs = src_rows[...]
    xt = tgt_rows[...]

    m16 = m16_ref[...]
    x_edge = jnp.concatenate(
        [ed_ref[...], xs[:, 72:88], xt[:, 88:104]], axis=1)
    h = jnp.dot(x_edge, rw1_ref[...], preferred_element_type=f32) + rb1_ref[...]
    h = _scaled_silu(_ln_mm(h, m16, rg1_ref[...], rbe1_ref[...]))
    h = jnp.dot(h, rw2_ref[...], preferred_element_type=f32) + rb2_ref[...]
    h = _scaled_silu(_ln_mm(h, m16, rg2_ref[...], rbe2_ref[...]))
    rad_full = jnp.dot(h, rw3_ref[...], preferred_element_type=f32) + rb3_ref[...]

    X = jnp.concatenate([xs[:, :72], xt[:, :72]], axis=1)
    wigi = wiginv_ref[...]
    rot_cat = None
    for j in range(K):
        d = jnp.dot(wigi, prot_ref[j], preferred_element_type=f32)
        v = jnp.dot(X, trot_ref[j], preferred_element_type=f32)
        rot_cat = d * v if rot_cat is None else rot_cat + d * v

    xr = rot_cat * rad_full
    y1 = jnp.dot(xr, wcat1_ref[...], preferred_element_type=f32) + bcat1_ref[...]
    alpha_feat = y1[:, :ALPHA_TOT]
    gating = y1[:, ALPHA_TOT:EXTRA_M0]
    hid_cat = jnp.dot(y1, g1_ref[...], preferred_element_type=f32)

    g = _scaled_silu(jnp.dot(hid_cat, tgx_ref[...], preferred_element_type=f32))
    y = jnp.dot(g, fgx_ref[...], preferred_element_type=f32)
    act_cat = jnp.concatenate(
        [_scaled_silu(gating), y[:, HIDDEN_CH:]], axis=1)

    y2 = jnp.dot(act_cat, wcat2_ref[...], preferred_element_type=f32) + bcat2_ref[...]
    vcat = jnp.dot(y2, g2_ref[...], preferred_element_type=f32)

    ln = _ln_mm(alpha_feat, m8_ref[...], ag8_ref[...], ab8_ref[...])
    alpha = jnp.dot(_smooth_leaky_relu(ln), adot_ref[...],
                    preferred_element_type=f32)

    wigf = wig_ref[...]
    msg = None
    for j in range(K):
        d = jnp.dot(wigf, pinv_ref[j], preferred_element_type=f32)
        v = jnp.dot(vcat, tinv_ref[j], preferred_element_type=f32)
        msg = d * v if msg is None else msg + d * v

    w = jnp.exp(alpha)
    wfull = jnp.dot(w, wexp_ref[...], preferred_element_type=f32)

    pk[...] = jnp.concatenate(
        [msg * wfull, w, jnp.zeros((te, PAD_W), f32)], axis=-1)

    for mi in range(te):
        a = accs[mi % 4]
        t = idx_ref[0, 1, mi]
        a[t, 0] = a[t, 0] + pk[mi]

    @pl.when(pl.program_id(1) == pl.num_programs(1) - 1)
    def _flush():
        out_ref[0] = ((acc0[...] + acc1[...]) + (acc2[...] + acc3[...]))


def _fused_edge_messages(idx2, edge_distance, wig2, wiginv2, node_tab, weights):
    E = wig2.shape[0]
    N = node_tab.shape[0]
    te = EDGE_TILE
    ns = E // te // 2
    grid = (2, ns)

    def row_spec(width):
        return pl.BlockSpec((te, width), lambda c, i: (c * ns + i, 0))

    in_specs = [
        pl.BlockSpec((1, 2, te), lambda c, i: (c * ns + i, 0, 0),
                     memory_space=pltpu.SMEM),
        row_spec(edge_distance.shape[1]),
        row_spec(wig2.shape[1]), row_spec(wiginv2.shape[1]),
        pl.BlockSpec(node_tab.shape, lambda c, i: (0, 0, 0)),
    ]
    in_specs += [pl.BlockSpec(w.shape, lambda c, i, n=w.ndim: (0,) * n)
                 for w in weights]

    out_shape = jax.ShapeDtypeStruct((2, N, 1, PACK_W), jnp.float32)
    out_specs = pl.BlockSpec((1, N, 1, PACK_W), lambda c, i: (c, 0, 0, 0))

    return pl.pallas_call(
        _fused_edge_kernel,
        out_shape=out_shape,
        grid=grid,
        in_specs=in_specs,
        out_specs=out_specs,
        scratch_shapes=[pltpu.VMEM((te, NODE_W), jnp.float32),
                        pltpu.VMEM((te, NODE_W), jnp.float32),
                        pltpu.VMEM((te, PACK_W), jnp.float32),
                        pltpu.VMEM((N, 1, PACK_W), jnp.float32),
                        pltpu.VMEM((N, 1, PACK_W), jnp.float32),
                        pltpu.VMEM((N, 1, PACK_W), jnp.float32),
                        pltpu.VMEM((N, 1, PACK_W), jnp.float32)],
        compiler_params=pltpu.CompilerParams(
            dimension_semantics=("parallel", "arbitrary"),
            vmem_limit_bytes=96 * 1024 * 1024),
    )(idx2, edge_distance, wig2, wiginv2, node_tab, *weights)


def _node_proj_kernel(acc_ref, w_ref, b_ref, wexp_ref, o_ref):
    acc = acc_ref[...]
    x = acc[:, :MSG_W]
    z = acc[:, MSG_W:MSG_W + NUM_HEADS]
    inv = 1.0 / (z + 1e-16)
    inv_full = jnp.dot(inv, wexp_ref[...], preferred_element_type=jnp.float32)
    o_ref[...] = jnp.dot(x * inv_full, w_ref[...],
                         preferred_element_type=jnp.float32) + b_ref[...]


def _node_divide_project(acc, wbd_pad, bias_row, wexp):
    N = acc.shape[0]
    return pl.pallas_call(
        _node_proj_kernel,
        out_shape=jax.ShapeDtypeStruct((N, PROJ_PACK_W), jnp.float32),
        grid=(1,),
        in_specs=[pl.BlockSpec((N, PACK_W), lambda i: (0, 0)),
                  pl.BlockSpec((MSG_W, PROJ_PACK_W), lambda i: (0, 0)),
                  pl.BlockSpec((1, PROJ_PACK_W), lambda i: (0, 0)),
                  pl.BlockSpec(wexp.shape, lambda i: (0, 0))],
        out_specs=pl.BlockSpec((N, PROJ_PACK_W), lambda i: (0, 0)),
    )(acc, wbd_pad, bias_row, wexp)


def kernel(x_emb, atomic_numbers, edge_distance, edge_index, wigner, wigner_inv,
           to_grid, from_grid, source_embedding, target_embedding,
           rad1_w1, rad1_b1, rad1_ln1_g, rad1_ln1_b, rad1_w2, rad1_b2,
           rad1_ln2_g, rad1_ln2_b, rad1_w3, rad1_b3,
           conv1_w0, conv1_b0, conv1_w1, conv1_w2,
           conv2_w0, conv2_b0, conv2_w1, conv2_w2,
           alpha_ln_g, alpha_ln_b, alpha_dot, proj_w, proj_b):
    E = edge_index.shape[1]
    N = x_emb.shape[0]
    te = EDGE_TILE
    src, tgt = edge_index[0], edge_index[1]

    node_tab = jnp.concatenate(
        [x_emb.reshape(N, K * SPHERE_CH),
         source_embedding[atomic_numbers],
         target_embedding[atomic_numbers],
         jnp.zeros((N, NODE_W - K * SPHERE_CH - 32), jnp.float32)],
        axis=1).reshape(N, 1, NODE_W)

    idx2 = jnp.stack([src, tgt], axis=0).reshape(2, E // te, te).transpose(1, 0, 2)

    wig2 = wigner.reshape(E, K * K)
    wiginv2 = wigner_inv.reshape(E, K * K)

    eye_h = jnp.eye(HIDDEN_CH, dtype=jnp.float32)
    tg_exp = jnp.kron(to_grid.T, eye_h)
    fg_exp = jnp.kron(from_grid, eye_h)

    radx = jnp.asarray(_RADX)
    rw3f = jnp.dot(rad1_w3, radx)
    rb3f = jnp.dot(rad1_b3.reshape(1, -1), radx)
    wcat1 = jnp.concatenate(
        [jnp.asarray(_sel_t(IDX0, C_IN1)) @ conv1_w0,
         jnp.asarray(_sel_t(PLUS1, C_IN1)) @ conv1_w1,
         jnp.asarray(_sel_t(MINUS1, C_IN1)) @ conv1_w1,
         jnp.asarray(_sel_t(PLUS2, C_IN1)) @ conv1_w2,
         jnp.asarray(_sel_t(MINUS2, C_IN1)) @ conv1_w2], axis=1)
    bcat1 = jnp.concatenate(
        [conv1_b0.reshape(1, -1), jnp.zeros((1, 96), jnp.float32)], axis=1)
    wcat2 = jnp.concatenate(
        [jnp.asarray(_sel_t(IDX0, HIDDEN_CH)) @ conv2_w0,
         jnp.asarray(_sel_t(PLUS1, HIDDEN_CH)) @ conv2_w1,
         jnp.asarray(_sel_t(MINUS1, HIDDEN_CH)) @ conv2_w1,
         jnp.asarray(_sel_t(PLUS2, HIDDEN_CH)) @ conv2_w2,
         jnp.asarray(_sel_t(MINUS2, HIDDEN_CH)) @ conv2_w2], axis=1)
    bcat2 = jnp.concatenate(
        [conv2_b0.reshape(1, -1), jnp.zeros((1, 96), jnp.float32)], axis=1)

    ag8 = jnp.tile(alpha_ln_g, 2).reshape(1, ALPHA_TOT)
    ab8 = jnp.tile(alpha_ln_b, 2).reshape(1, ALPHA_TOT)
    adot_bd = jnp.zeros((ALPHA_TOT, NUM_HEADS), jnp.float32)
    adot_bd = adot_bd.at[:ATTN_ALPHA_CH, 0].set(alpha_dot[0])
    adot_bd = adot_bd.at[ATTN_ALPHA_CH:, 1].set(alpha_dot[1])

    weights = [
        rad1_w1, rad1_b1.reshape(1, -1),
        rad1_ln1_g.reshape(1, -1), rad1_ln1_b.reshape(1, -1),
        rad1_w2, rad1_b2.reshape(1, -1),
        rad1_ln2_g.reshape(1, -1), rad1_ln2_b.reshape(1, -1),
        rw3f, rb3f,
        wcat1, bcat1, jnp.asarray(_G1),
        tg_exp, fg_exp,
        wcat2, bcat2, jnp.asarray(_G2),
        jnp.asarray(_M16), jnp.asarray(_M8), ag8, ab8, adot_bd,
        jnp.asarray(_WEXP),
        jnp.asarray(_PROT), jnp.asarray(_TROT),
        jnp.asarray(_PINV), jnp.asarray(_TINV),
    ]

    parts = _fused_edge_messages(idx2, edge_distance, wig2, wiginv2,
                                 node_tab, weights)
    acc = (parts[0] + parts[1]).reshape(N, PACK_W)

    w_per = jnp.transpose(proj_w[L_PER_COEF], (0, 2, 1))
    eye_k = jnp.eye(K, dtype=jnp.float32)
    wbd = (eye_k[:, None, :, None] * w_per[:, :, None, :]).reshape(MSG_W, PROJ_W)
    wbd_pad = jnp.zeros((MSG_W, PROJ_PACK_W), jnp.float32).at[:, :PROJ_W].set(wbd)
    bias_row = jnp.zeros((1, PROJ_PACK_W), jnp.float32).at[0, :OUTPUT_CH].set(proj_b)
    out = _node_divide_project(acc, wbd_pad, bias_row, jnp.asarray(_WEXP))[:, :PROJ_W]
    return out.reshape(N, K, OUTPUT_CH)


# speedup vs baseline: 2.0911x; 1.0071x over previous
import math
import numpy as np

import jax
import jax.numpy as jnp
from jax.experimental import pallas as pl
from jax.experimental.pallas import tpu as pltpu

LMAX = 2
MMAX = 2
K = (LMAX + 1) ** 2
SPHERE_CH = 8
HIDDEN_CH = 8
NUM_HEADS = 2
ATTN_ALPHA_CH = 4
ATTN_VALUE_CH = 4
OUTPUT_CH = 8
SILU_SCALE = 1.0 / 0.6

C_IN1 = 2 * SPHERE_CH
ALPHA_TOT = NUM_HEADS * ATTN_ALPHA_CH
VALUE_TOT = NUM_HEADS * ATTN_VALUE_CH
EXTRA_M0 = ALPHA_TOT + HIDDEN_CH

EDGE_TILE = 2048

MSG_W = K * VALUE_TOT
PACK_W = 128
PAD_W = PACK_W - MSG_W - NUM_HEADS

PROJ_W = K * OUTPUT_CH
PROJ_PACK_W = 128

NODE_W = 128
X_W = K * C_IN1

IDX0 = [0, 2, 6]
PLUS1, MINUS1 = [3, 7], [1, 5]
PLUS2, MINUS2 = [8], [4]

L_PER_COEF = np.concatenate([[l] * (2 * l + 1) for l in range(LMAX + 1)]).astype(np.int32)


def _expansion_mats():
    prot = np.zeros((K, K * K, X_W), np.float32)
    trot = np.zeros((K, X_W, X_W), np.float32)
    pinv = np.zeros((K, K * K, MSG_W), np.float32)
    tinv = np.zeros((K, MSG_W, MSG_W), np.float32)
    for j in range(K):
        for k in range(K):
            prot[j, j * K + k, k * C_IN1:(k + 1) * C_IN1] = 1.0
            pinv[j, j * K + k, k * VALUE_TOT:(k + 1) * VALUE_TOT] = 1.0
            for c in range(SPHERE_CH):
                trot[j, j * SPHERE_CH + c, k * C_IN1 + c] = 1.0
                trot[j, K * SPHERE_CH + j * SPHERE_CH + c,
                     k * C_IN1 + SPHERE_CH + c] = 1.0
            for c in range(VALUE_TOT):
                tinv[j, j * VALUE_TOT + c, k * VALUE_TOT + c] = 1.0
    return prot, trot, pinv, tinv


def _sel_t(coefs, ch):
    s = np.zeros((K * ch, len(coefs) * ch), np.float32)
    for t, i in enumerate(coefs):
        for c in range(ch):
            s[i * ch + c, t * ch + c] = 1.0
    return s


def _gather_mat(ch, extra):
    n0, n1, n2 = len(IDX0) * ch, len(PLUS1) * ch, len(PLUS2) * ch
    cols = extra + n0 + 2 * (2 * n1) + 2 * (2 * n2)
    g = np.zeros((cols, K * ch), np.float32)
    o = extra
    for t, i in enumerate(IDX0):
        for c in range(ch):
            g[o + t * ch + c, i * ch + c] = 1.0
    o += n0
    for t, i in enumerate(PLUS1):
        for c in range(ch):
            g[o + t * ch + c, i * ch + c] = 1.0
    o += n1
    for t, i in enumerate(MINUS1):
        for c in range(ch):
            g[o + t * ch + c, i * ch + c] = 1.0
    o += n1
    for t, i in enumerate(MINUS1):
        for c in range(ch):
            g[o + t * ch + c, i * ch + c] = 1.0
    o += n1
    for t, i in enumerate(PLUS1):
        for c in range(ch):
            g[o + t * ch + c, i * ch + c] = -1.0
    o += n1
    for t, i in enumerate(PLUS2):
        for c in range(ch):
            g[o + t * ch + c, i * ch + c] = 1.0
    o += n2
    for t, i in enumerate(MINUS2):
        for c in range(ch):
            g[o + t * ch + c, i * ch + c] = 1.0
    o += n2
    for t, i in enumerate(MINUS2):
        for c in range(ch):
            g[o + t * ch + c, i * ch + c] = 1.0
    o += n2
    for t, i in enumerate(PLUS2):
        for c in range(ch):
            g[o + t * ch + c, i * ch + c] = -1.0
    return g


def _rad_expand():
    r = np.zeros((96, X_W), np.float32)
    for t, i in enumerate(IDX0):
        for c in range(C_IN1):
            r[t * C_IN1 + c, i * C_IN1 + c] = 1.0
    for t in range(len(PLUS1)):
        for c in range(C_IN1):
            r[48 + t * C_IN1 + c, PLUS1[t] * C_IN1 + c] = 1.0
            r[48 + t * C_IN1 + c, MINUS1[t] * C_IN1 + c] = 1.0
    for c in range(C_IN1):
        r[80 + c, PLUS2[0] * C_IN1 + c] = 1.0
        r[80 + c, MINUS2[0] * C_IN1 + c] = 1.0
    return r


_PROT, _TROT, _PINV, _TINV = _expansion_mats()
_G1 = _gather_mat(HIDDEN_CH, EXTRA_M0)
_G2 = _gather_mat(VALUE_TOT, 0)
_RADX = _rad_expand()
_M16 = np.full((16, 16), 1.0 / 16.0, np.float32)
_M8 = np.kron(np.eye(2, dtype=np.float32),
              np.full((4, 4), 0.25, np.float32))
_WEXP = np.zeros((NUM_HEADS, MSG_W), np.float32)
for _h in range(NUM_HEADS):
    for _k in range(K):
        for _c in range(ATTN_VALUE_CH):
            _WEXP[_h, _k * VALUE_TOT + _h * ATTN_VALUE_CH + _c] = 1.0


def _scaled_silu(x):
    return x * jax.nn.sigmoid(x) * SILU_SCALE


def _smooth_leaky_relu(x, alpha=0.2):
    return ((1.0 + alpha) / 2.0) * x + ((1.0 - alpha) / 2.0) * x * (2.0 * jax.nn.sigmoid(x) - 1.0)


def _ln_mm(x, mmat, g, b, eps=1e-5):
    mu = jnp.dot(x, mmat, preferred_element_type=jnp.float32)
    d = x - mu
    var = jnp.dot(d * d, mmat, preferred_element_type=jnp.float32)
    return d * jax.lax.rsqrt(var + eps) * g + b


def _fused_edge_kernel(
        idx_ref, ed_ref, wig_ref, wiginv_ref, ntab_ref,
        rw1_ref, rb1_ref, rg1_ref, rbe1_ref,
        rw2_ref, rb2_ref, rg2_ref, rbe2_ref,
        rw3_ref, rb3_ref,
        wcat1_ref, bcat1_ref, g1_ref,
        tgx_ref, fgx_ref,
        wcat2_ref, bcat2_ref, g2_ref,
        m16_ref, m8_ref, ag8_ref, ab8_ref, adot_ref, wexp_ref,
        prot_ref, trot_ref, pinv_ref, tinv_ref,
        out_ref, src_rows, tgt_rows, pk, acc0, acc1, acc2, acc3):
    f32 = jnp.float32
    te = EDGE_TILE
    accs = [acc0, acc1, acc2, acc3]

    @pl.when(pl.program_id(1) == 0)
    def _init():
        for a in accs:
            a[...] = jnp.zeros_like(a)

    for mi in range(te):
        src_rows[mi] = ntab_ref[idx_ref[0, 0, mi], 0]
        tgt_rows[mi] = ntab_ref[idx_ref[0, 1, mi], 0]
    xs = src_rows[...]
    xt = tgt_rows[...]

    m16 = m16_ref[...]
    x_edge = jnp.concatenate(
        [ed_ref[...], xs[:, 72:88], xt[:, 88:104]], axis=1)
    h = jnp.dot(x_edge, rw1_ref[...], preferred_element_type=f32) + rb1_ref[...]
    h = _scaled_silu(_ln_mm(h, m16, rg1_ref[...], rbe1_ref[...]))
    h = jnp.dot(h, rw2_ref[...], preferred_element_type=f32) + rb2_ref[...]
    h = _scaled_silu(_ln_mm(h, m16, rg2_ref[...], rbe2_ref[...]))
    rad_full = jnp.dot(h, rw3_ref[...], preferred_element_type=f32) + rb3_ref[...]

    X = jnp.concatenate([xs[:, :72], xt[:, :72]], axis=1)
    wigi = wiginv_ref[...]
    rot_cat = None
    for j in range(K):
        d = jnp.dot(wigi, prot_ref[j], preferred_element_type=f32)
        v = jnp.dot(X, trot_ref[j], preferred_element_type=f32)
        rot_cat = d * v if rot_cat is None else rot_cat + d * v

    xr = rot_cat * rad_full
    y1 = jnp.dot(xr, wcat1_ref[...], preferred_element_type=f32) + bcat1_ref[...]
    alpha_feat = y1[:, :ALPHA_TOT]
    gating = y1[:, ALPHA_TOT:EXTRA_M0]
    hid_cat = jnp.dot(y1, g1_ref[...], preferred_element_type=f32)

    g = _scaled_silu(jnp.dot(hid_cat, tgx_ref[...], preferred_element_type=f32))
    y = jnp.dot(g, fgx_ref[...], preferred_element_type=f32)
    act_cat = jnp.concatenate(
        [_scaled_silu(gating), y[:, HIDDEN_CH:]], axis=1)

    y2 = jnp.dot(act_cat, wcat2_ref[...], preferred_element_type=f32) + bcat2_ref[...]
    vcat = jnp.dot(y2, g2_ref[...], preferred_element_type=f32)

    ln = _ln_mm(alpha_feat, m8_ref[...], ag8_ref[...], ab8_ref[...])
    alpha = jnp.dot(_smooth_leaky_relu(ln), adot_ref[...],
                    preferred_element_type=f32)

    wigf = wig_ref[...]
    msg = None
    for j in range(K):
        d = jnp.dot(wigf, pinv_ref[j], preferred_element_type=f32)
        v = jnp.dot(vcat, tinv_ref[j], preferred_element_type=f32)
        msg = d * v if msg is None else msg + d * v

    w = jnp.exp(alpha)
    wfull = jnp.dot(w, wexp_ref[...], preferred_element_type=f32)

    pk[...] = jnp.concatenate(
        [msg * wfull, w, jnp.zeros((te, PAD_W), f32)], axis=-1)

    for mi in range(te):
        a = accs[mi % 4]
        t = idx_ref[0, 1, mi]
        a[t, 0] = a[t, 0] + pk[mi]

    @pl.when(pl.program_id(1) == pl.num_programs(1) - 1)
    def _flush():
        out_ref[0] = ((acc0[...] + acc1[...]) + (acc2[...] + acc3[...]))


def _fused_edge_messages(idx2, edge_distance, wig2, wiginv2, node_tab, weights):
    E = wig2.shape[0]
    N = node_tab.shape[0]
    te = EDGE_TILE
    ns = E // te // 2
    grid = (2, ns)

    def row_spec(width):
        return pl.BlockSpec((te, width), lambda c, i: (c * ns + i, 0))

    in_specs = [
        pl.BlockSpec((1, 2, te), lambda c, i: (c * ns + i, 0, 0),
                     memory_space=pltpu.SMEM),
        row_spec(edge_distance.shape[1]),
        row_spec(wig2.shape[1]), row_spec(wiginv2.shape[1]),
        pl.BlockSpec(node_tab.shape, lambda c, i: (0, 0, 0)),
    ]
    in_specs += [pl.BlockSpec(w.shape, lambda c, i, n=w.ndim: (0,) * n)
                 for w in weights]

    out_shape = jax.ShapeDtypeStruct((2, N, 1, PACK_W), jnp.float32)
    out_specs = pl.BlockSpec((1, N, 1, PACK_W), lambda c, i: (c, 0, 0, 0))

    return pl.pallas_call(
        _fused_edge_kernel,
        out_shape=out_shape,
        grid=grid,
        in_specs=in_specs,
        out_specs=out_specs,
        scratch_shapes=[pltpu.VMEM((te, NODE_W), jnp.float32),
                        pltpu.VMEM((te, NODE_W), jnp.float32),
                        pltpu.VMEM((te, PACK_W), jnp.float32),
                        pltpu.VMEM((N, 1, PACK_W), jnp.float32),
                        pltpu.VMEM((N, 1, PACK_W), jnp.float32),
                        pltpu.VMEM((N, 1, PACK_W), jnp.float32),
                        pltpu.VMEM((N, 1, PACK_W), jnp.float32)],
        compiler_params=pltpu.CompilerParams(
            dimension_semantics=("parallel", "arbitrary"),
            vmem_limit_bytes=96 * 1024 * 1024),
    )(idx2, edge_distance, wig2, wiginv2, node_tab, *weights)


def _node_proj_kernel(acc_ref, w_ref, b_ref, wexp_ref, o_ref):
    acc = acc_ref[...]
    x = acc[:, :MSG_W]
    z = acc[:, MSG_W:MSG_W + NUM_HEADS]
    inv = 1.0 / (z + 1e-16)
    inv_full = jnp.dot(inv, wexp_ref[...], preferred_element_type=jnp.float32)
    o_ref[...] = jnp.dot(x * inv_full, w_ref[...],
                         preferred_element_type=jnp.float32) + b_ref[...]


def _node_divide_project(acc, wbd_pad, bias_row, wexp):
    N = acc.shape[0]
    return pl.pallas_call(
        _node_proj_kernel,
        out_shape=jax.ShapeDtypeStruct((N, PROJ_PACK_W), jnp.float32),
        grid=(1,),
        in_specs=[pl.BlockSpec((N, PACK_W), lambda i: (0, 0)),
                  pl.BlockSpec((MSG_W, PROJ_PACK_W), lambda i: (0, 0)),
                  pl.BlockSpec((1, PROJ_PACK_W), lambda i: (0, 0)),
                  pl.BlockSpec(wexp.shape, lambda i: (0, 0))],
        out_specs=pl.BlockSpec((N, PROJ_PACK_W), lambda i: (0, 0)),
    )(acc, wbd_pad, bias_row, wexp)


def kernel(x_emb, atomic_numbers, edge_distance, edge_index, wigner, wigner_inv,
           to_grid, from_grid, source_embedding, target_embedding,
           rad1_w1, rad1_b1, rad1_ln1_g, rad1_ln1_b, rad1_w2, rad1_b2,
           rad1_ln2_g, rad1_ln2_b, rad1_w3, rad1_b3,
           conv1_w0, conv1_b0, conv1_w1, conv1_w2,
           conv2_w0, conv2_b0, conv2_w1, conv2_w2,
           alpha_ln_g, alpha_ln_b, alpha_dot, proj_w, proj_b):
    E = edge_index.shape[1]
    N = x_emb.shape[0]
    te = EDGE_TILE
    src, tgt = edge_index[0], edge_index[1]

    node_tab = jnp.concatenate(
        [x_emb.reshape(N, K * SPHERE_CH),
         source_embedding[atomic_numbers],
         target_embedding[atomic_numbers],
         jnp.zeros((N, NODE_W - K * SPHERE_CH - 32), jnp.float32)],
        axis=1).reshape(N, 1, NODE_W)

    idx2 = jnp.stack([src, tgt], axis=0).reshape(2, E // te, te).transpose(1, 0, 2)

    wig2 = wigner.reshape(E, K * K)
    wiginv2 = wigner_inv.reshape(E, K * K)

    eye_h = jnp.eye(HIDDEN_CH, dtype=jnp.float32)
    tg_exp = jnp.kron(to_grid.T, eye_h)
    fg_exp = jnp.kron(from_grid, eye_h)

    radx = jnp.asarray(_RADX)
    rw3f = jnp.dot(rad1_w3, radx)
    rb3f = jnp.dot(rad1_b3.reshape(1, -1), radx)
    wcat1 = jnp.concatenate(
        [jnp.asarray(_sel_t(IDX0, C_IN1)) @ conv1_w0,
         jnp.asarray(_sel_t(PLUS1, C_IN1)) @ conv1_w1,
         jnp.asarray(_sel_t(MINUS1, C_IN1)) @ conv1_w1,
         jnp.asarray(_sel_t(PLUS2, C_IN1)) @ conv1_w2,
         jnp.asarray(_sel_t(MINUS2, C_IN1)) @ conv1_w2], axis=1)
    bcat1 = jnp.concatenate(
        [conv1_b0.reshape(1, -1), jnp.zeros((1, 96), jnp.float32)], axis=1)
    wcat2 = jnp.concatenate(
        [jnp.asarray(_sel_t(IDX0, HIDDEN_CH)) @ conv2_w0,
         jnp.asarray(_sel_t(PLUS1, HIDDEN_CH)) @ conv2_w1,
         jnp.asarray(_sel_t(MINUS1, HIDDEN_CH)) @ conv2_w1,
         jnp.asarray(_sel_t(PLUS2, HIDDEN_CH)) @ conv2_w2,
         jnp.asarray(_sel_t(MINUS2, HIDDEN_CH)) @ conv2_w2], axis=1)
    bcat2 = jnp.concatenate(
        [conv2_b0.reshape(1, -1), jnp.zeros((1, 96), jnp.float32)], axis=1)

    ag8 = jnp.tile(alpha_ln_g, 2).reshape(1, ALPHA_TOT)
    ab8 = jnp.tile(alpha_ln_b, 2).reshape(1, ALPHA_TOT)
    adot_bd = jnp.zeros((ALPHA_TOT, NUM_HEADS), jnp.float32)
    adot_bd = adot_bd.at[:ATTN_ALPHA_CH, 0].set(alpha_dot[0])
    adot_bd = adot_bd.at[ATTN_ALPHA_CH:, 1].set(alpha_dot[1])

    weights = [
        rad1_w1, rad1_b1.reshape(1, -1),
        rad1_ln1_g.reshape(1, -1), rad1_ln1_b.reshape(1, -1),
        rad1_w2, rad1_b2.reshape(1, -1),
        rad1_ln2_g.reshape(1, -1), rad1_ln2_b.reshape(1, -1),
        rw3f, rb3f,
        wcat1, bcat1, jnp.asarray(_G1),
        tg_exp, fg_exp,
        wcat2, bcat2, jnp.asarray(_G2),
        jnp.asarray(_M16), jnp.asarray(_M8), ag8, ab8, adot_bd,
        jnp.asarray(_WEXP),
        jnp.asarray(_PROT), jnp.asarray(_TROT),
        jnp.asarray(_PINV), jnp.asarray(_TINV),
    ]

    parts = _fused_edge_messages(idx2, edge_distance, wig2, wiginv2,
                                 node_tab, weights)
    acc = (parts[0] + parts[1]).reshape(N, PACK_W)

    w_per = jnp.transpose(proj_w[L_PER_COEF], (0, 2, 1))
    eye_k = jnp.eye(K, dtype=jnp.float32)
    wbd = (eye_k[:, None, :, None] * w_per[:, :, None, :]).reshape(MSG_W, PROJ_W)
    wbd_pad = jnp.zeros((MSG_W, PROJ_PACK_W), jnp.float32).at[:, :PROJ_W].set(wbd)
    bias_row = jnp.zeros((1, PROJ_PACK_W), jnp.float32).at[0, :OUTPUT_CH].set(proj_b)
    out = _node_divide_project(acc, wbd_pad, bias_row, jnp.asarray(_WEXP))[:, :PROJ_W]
    return out.reshape(N, K, OUTPUT_CH)
